# baseline (speedup 1.0000x reference)
import functools

import jax
import jax.numpy as jnp
from jax.experimental import pallas as pl
from jax.experimental.pallas import tpu as pltpu

P = 32
KSEL = 8
D = 2048
OFF_BIAS = 0.01
OFF_VAR = 0.01


def _fused_kernel(x_ref, keys_ref, biases_ref, sel_ref, w_ref, loss_ref, *, n_tok):
    b = pl.program_id(0)
    t = pl.program_id(1)

    x = x_ref[0]
    keys = keys_ref[...]
    raw = jax.lax.dot_general(
        keys, x, (((1,), (1,)), ((), ())),
        preferred_element_type=jnp.float32,
    )
    scores = raw + biases_ref[...]

    tb = scores.shape[1]
    row_ids = jax.lax.broadcasted_iota(jnp.int32, (P, tb), 0)

    work = scores
    vals = []
    idxs = []
    for _ in range(KSEL):
        mx = jnp.max(work, axis=0, keepdims=True)
        hit = work == mx
        idx = jnp.min(jnp.where(hit, row_ids, P), axis=0, keepdims=True)
        vals.append(mx)
        idxs.append(idx)
        work = jnp.where(row_ids == idx, -jnp.inf, work)

    sel_vals = jnp.concatenate(vals, axis=0)
    sel_idx = jnp.concatenate(idxs, axis=0)
    ex = jnp.exp(sel_vals - sel_vals[0:1])
    w = ex / jnp.sum(ex, axis=0, keepdims=True)

    sel_ref[0] = sel_idx
    w_ref[0] = w

    sum_p = jnp.sum(raw, axis=0)
    sumsq = jnp.sum(raw * raw, axis=0)
    m = sum_p * (1.0 / P)
    var = (sumsq - P * m * m) * (1.0 / (P - 1))
    term2 = (1.0 - jnp.sqrt(var)) ** 2
    part = (OFF_BIAS * jnp.sum(m * m) + OFF_VAR * jnp.sum(term2)) * (1.0 / n_tok)

    @pl.when((b == 0) & (t == 0))
    def _():
        loss_ref[...] = jnp.zeros_like(loss_ref)

    loss_ref[...] += part.reshape(1, 1)


def kernel(tensor, keys, biases, partitions, connectome_biases):
    del partitions, connectome_biases
    B, T, _ = tensor.shape
    tb = 512
    nb = T // tb
    n_tok = B * T

    sel, w, loss = pl.pallas_call(
        functools.partial(_fused_kernel, n_tok=n_tok),
        grid=(B, nb),
        in_specs=[
            pl.BlockSpec((1, tb, D), lambda b, t: (b, t, 0)),
            pl.BlockSpec((P, D), lambda b, t: (0, 0)),
            pl.BlockSpec((P, 1), lambda b, t: (0, 0)),
        ],
        out_specs=[
            pl.BlockSpec((1, KSEL, tb), lambda b, t: (b, 0, t)),
            pl.BlockSpec((1, KSEL, tb), lambda b, t: (b, 0, t)),
            pl.BlockSpec((1, 1), lambda b, t: (0, 0)),
        ],
        out_shape=[
            jax.ShapeDtypeStruct((B, KSEL, T), jnp.int32),
            jax.ShapeDtypeStruct((B, KSEL, T), jnp.float32),
            jax.ShapeDtypeStruct((1, 1), jnp.float32),
        ],
        compiler_params=pltpu.CompilerParams(
            dimension_semantics=("arbitrary", "arbitrary"),
        ),
    )(tensor, keys, biases.reshape(P, 1))

    return sel, w, loss[0, 0]


# speedup vs baseline: 1.2146x; 1.2146x over previous
import functools

import jax
import jax.numpy as jnp
from jax.experimental import pallas as pl
from jax.experimental.pallas import tpu as pltpu

P = 32
KSEL = 8
D = 2048
OFF_BIAS = 0.01
OFF_VAR = 0.01


def _fused_kernel(x_ref, keys_ref, biases_ref, sel_ref, w_ref, loss_ref, *, n_tok):
    b = pl.program_id(0)
    t = pl.program_id(1)

    x = x_ref[0]
    keys = keys_ref[...]
    raw = jax.lax.dot_general(
        keys, x, (((1,), (1,)), ((), ())),
        preferred_element_type=jnp.float32,
    )
    scores = raw + biases_ref[...]

    tb = scores.shape[1]
    row_ids = jax.lax.broadcasted_iota(jnp.int32, (P, tb), 0)

    work = scores
    vals = []
    idxs = []
    for _ in range(KSEL):
        mx = jnp.max(work, axis=0, keepdims=True)
        hit = work == mx
        idx = jnp.min(jnp.where(hit, row_ids, P), axis=0, keepdims=True)
        vals.append(mx)
        idxs.append(idx)
        work = jnp.where(row_ids == idx, -jnp.inf, work)

    sel_vals = jnp.concatenate(vals, axis=0)
    sel_idx = jnp.concatenate(idxs, axis=0)
    ex = jnp.exp(sel_vals - sel_vals[0:1])
    w = ex / jnp.sum(ex, axis=0, keepdims=True)

    sel_ref[0] = sel_idx
    w_ref[0] = w

    sum_p = jnp.sum(raw, axis=0)
    sumsq = jnp.sum(raw * raw, axis=0)
    m = sum_p * (1.0 / P)
    var = (sumsq - P * m * m) * (1.0 / (P - 1))
    term2 = (1.0 - jnp.sqrt(var)) ** 2
    part = (OFF_BIAS * jnp.sum(m * m) + OFF_VAR * jnp.sum(term2)) * (1.0 / n_tok)

    @pl.when((b == 0) & (t == 0))
    def _():
        loss_ref[...] = jnp.zeros_like(loss_ref)

    loss_ref[...] += part.reshape(1, 1)


def kernel(tensor, keys, biases, partitions, connectome_biases):
    del partitions, connectome_biases
    B, T, _ = tensor.shape
    tb = 4096
    nb = T // tb
    n_tok = B * T

    sel, w, loss = pl.pallas_call(
        functools.partial(_fused_kernel, n_tok=n_tok),
        grid=(B, nb),
        in_specs=[
            pl.BlockSpec((1, tb, D), lambda b, t: (b, t, 0)),
            pl.BlockSpec((P, D), lambda b, t: (0, 0)),
            pl.BlockSpec((P, 1), lambda b, t: (0, 0)),
        ],
        out_specs=[
            pl.BlockSpec((1, KSEL, tb), lambda b, t: (b, 0, t)),
            pl.BlockSpec((1, KSEL, tb), lambda b, t: (b, 0, t)),
            pl.BlockSpec((1, 1), lambda b, t: (0, 0)),
        ],
        out_shape=[
            jax.ShapeDtypeStruct((B, KSEL, T), jnp.int32),
            jax.ShapeDtypeStruct((B, KSEL, T), jnp.float32),
            jax.ShapeDtypeStruct((1, 1), jnp.float32),
        ],
        compiler_params=pltpu.CompilerParams(
            dimension_semantics=("arbitrary", "arbitrary"),
        ),
    )(tensor, keys, biases.reshape(P, 1))

    return sel, w, loss[0, 0]


# speedup vs baseline: 1.2327x; 1.0149x over previous
import functools

import jax
import jax.numpy as jnp
from jax.experimental import pallas as pl
from jax.experimental.pallas import tpu as pltpu

P = 32
KSEL = 8
D = 2048
OFF_BIAS = 0.01
OFF_VAR = 0.01


def _fused_kernel(x_ref, keys_ref, biases_ref, sel_ref, w_ref, loss_ref, *, n_tok):
    b = pl.program_id(0)
    t = pl.program_id(1)

    x = x_ref[0]
    keys = keys_ref[...]
    raw = jax.lax.dot_general(
        keys, x, (((1,), (1,)), ((), ())),
        preferred_element_type=jnp.float32,
    )
    scores = raw + biases_ref[...]

    tb = scores.shape[1]
    row_ids = jax.lax.broadcasted_iota(jnp.int32, (P, tb), 0)

    bits = jax.lax.bitcast_convert_type(scores, jnp.int32)
    sortable = bits ^ ((bits >> 31) & jnp.int32(0x7FFFFFFF))
    work = (sortable & jnp.int32(~31)) | (31 - row_ids)

    keys_out = []
    for _ in range(KSEL):
        mx = jnp.max(work, axis=0, keepdims=True)
        keys_out.append(mx)
        work = jnp.where(work == mx, jnp.int32(-(2**31)), work)

    mxs = jnp.concatenate(keys_out, axis=0)
    sel_idx = 31 - (mxs & 31)
    vbits = (mxs & jnp.int32(~31)) | 16
    vbits = vbits ^ ((vbits >> 31) & jnp.int32(0x7FFFFFFF))
    sel_vals = jax.lax.bitcast_convert_type(vbits, jnp.float32)
    ex = jnp.exp(sel_vals - sel_vals[0:1])
    w = ex / jnp.sum(ex, axis=0, keepdims=True)

    sel_ref[0] = sel_idx
    w_ref[0] = w

    sum_p = jnp.sum(raw, axis=0)
    sumsq = jnp.sum(raw * raw, axis=0)
    m = sum_p * (1.0 / P)
    var = (sumsq - P * m * m) * (1.0 / (P - 1))
    term2 = (1.0 - jnp.sqrt(var)) ** 2
    part = (OFF_BIAS * jnp.sum(m * m) + OFF_VAR * jnp.sum(term2)) * (1.0 / n_tok)

    @pl.when((b == 0) & (t == 0))
    def _():
        loss_ref[...] = jnp.zeros_like(loss_ref)

    loss_ref[...] += part.reshape(1, 1)


def kernel(tensor, keys, biases, partitions, connectome_biases):
    del partitions, connectome_biases
    B, T, _ = tensor.shape
    tb = 2048
    nb = T // tb
    n_tok = B * T

    sel, w, loss = pl.pallas_call(
        functools.partial(_fused_kernel, n_tok=n_tok),
        grid=(B, nb),
        in_specs=[
            pl.BlockSpec((1, tb, D), lambda b, t: (b, t, 0)),
            pl.BlockSpec((P, D), lambda b, t: (0, 0)),
            pl.BlockSpec((P, 1), lambda b, t: (0, 0)),
        ],
        out_specs=[
            pl.BlockSpec((1, KSEL, tb), lambda b, t: (b, 0, t)),
            pl.BlockSpec((1, KSEL, tb), lambda b, t: (b, 0, t)),
            pl.BlockSpec((1, 1), lambda b, t: (0, 0)),
        ],
        out_shape=[
            jax.ShapeDtypeStruct((B, KSEL, T), jnp.int32),
            jax.ShapeDtypeStruct((B, KSEL, T), jnp.float32),
            jax.ShapeDtypeStruct((1, 1), jnp.float32),
        ],
        compiler_params=pltpu.CompilerParams(
            dimension_semantics=("arbitrary", "arbitrary"),
        ),
    )(tensor, keys, biases.reshape(P, 1))

    return sel, w, loss[0, 0]


# speedup vs baseline: 1.2629x; 1.0245x over previous
import functools

import jax
import jax.numpy as jnp
from jax.experimental import pallas as pl
from jax.experimental.pallas import tpu as pltpu

P = 32
KSEL = 8
D = 2048
OFF_BIAS = 0.01
OFF_VAR = 0.01


def _fused_kernel(x_ref, keys_ref, biases_ref, sel_ref, w_ref, loss_ref, *, n_tok):
    b = pl.program_id(0)
    t = pl.program_id(1)

    x = x_ref[0]
    keys = keys_ref[...]
    raw = jax.lax.dot_general(
        keys, x, (((1,), (1,)), ((), ())),
        preferred_element_type=jnp.float32,
    )
    scores = raw + biases_ref[...].reshape(P, 1)

    tb = scores.shape[1]
    row_ids = jax.lax.broadcasted_iota(jnp.int32, (P, tb), 0)

    work = scores
    vals = []
    idxs = []
    for _ in range(KSEL):
        mx = jnp.max(work, axis=0, keepdims=True)
        hit = work == mx
        idx = jnp.min(jnp.where(hit, row_ids, P), axis=0, keepdims=True)
        vals.append(mx)
        idxs.append(idx)
        work = jnp.where(row_ids == idx, -jnp.inf, work)

    sel_vals = jnp.concatenate(vals, axis=0)
    sel_idx = jnp.concatenate(idxs, axis=0)
    ex = jnp.exp(sel_vals - sel_vals[0:1])
    w = ex / jnp.sum(ex, axis=0, keepdims=True)

    sel_ref[0] = sel_idx
    w_ref[0] = w

    sum_p = jnp.sum(raw, axis=0)
    sumsq = jnp.sum(raw * raw, axis=0)
    m = sum_p * (1.0 / P)
    var = (sumsq - P * m * m) * (1.0 / (P - 1))
    term2 = (1.0 - jnp.sqrt(var)) ** 2
    part = (OFF_BIAS * jnp.sum(m * m) + OFF_VAR * jnp.sum(term2)) * (1.0 / n_tok)

    @pl.when((b == 0) & (t == 0))
    def _():
        loss_ref[...] = jnp.zeros_like(loss_ref)

    loss_ref[...] += part.reshape(1, 1)


def kernel(tensor, keys, biases, partitions, connectome_biases):
    del partitions, connectome_biases
    B, T, _ = tensor.shape
    tb = 2048
    nb = T // tb
    n_tok = B * T

    sel, w, loss = pl.pallas_call(
        functools.partial(_fused_kernel, n_tok=n_tok),
        grid=(B, nb),
        in_specs=[
            pl.BlockSpec((1, tb, D), lambda b, t: (b, t, 0)),
            pl.BlockSpec((P, D), lambda b, t: (0, 0)),
            pl.BlockSpec((P,), lambda b, t: (0,)),
        ],
        out_specs=[
            pl.BlockSpec((1, KSEL, tb), lambda b, t: (b, 0, t)),
            pl.BlockSpec((1, KSEL, tb), lambda b, t: (b, 0, t)),
            pl.BlockSpec((1, 1), lambda b, t: (0, 0)),
        ],
        out_shape=[
            jax.ShapeDtypeStruct((B, KSEL, T), jnp.int32),
            jax.ShapeDtypeStruct((B, KSEL, T), jnp.float32),
            jax.ShapeDtypeStruct((1, 1), jnp.float32),
        ],
        compiler_params=pltpu.CompilerParams(
            dimension_semantics=("arbitrary", "arbitrary"),
        ),
    )(tensor, keys, biases)

    return sel, w, loss[0, 0]


# speedup vs baseline: 1.2670x; 1.0032x over previous
import functools

import jax
import jax.numpy as jnp
from jax.experimental import pallas as pl
from jax.experimental.pallas import tpu as pltpu

P = 32
KSEL = 8
D = 2048
OFF_BIAS = 0.01
OFF_VAR = 0.01


def _fused_kernel(x_ref, keys_ref, biases_ref, sel_ref, w_ref, loss_ref, *,
                  n_tok, n_half):
    b = pl.program_id(0)
    t = pl.program_id(1)

    keys = keys_ref[...]
    biases = biases_ref[...].reshape(P, 1)
    tb = x_ref.shape[1]
    hw = tb // n_half

    part = jnp.zeros((1, 1), jnp.float32)
    for h in range(n_half):
        x = x_ref[0, pl.ds(h * hw, hw), :]
        raw = jax.lax.dot_general(
            keys, x, (((1,), (1,)), ((), ())),
            preferred_element_type=jnp.float32,
        )
        scores = raw + biases

        row_ids = jax.lax.broadcasted_iota(jnp.int32, (P, hw), 0)
        work = scores
        vals = []
        idxs = []
        for _ in range(KSEL):
            mx = jnp.max(work, axis=0, keepdims=True)
            hit = work == mx
            idx = jnp.min(jnp.where(hit, row_ids, P), axis=0, keepdims=True)
            vals.append(mx)
            idxs.append(idx)
            work = jnp.where(row_ids == idx, -jnp.inf, work)

        sel_vals = jnp.concatenate(vals, axis=0)
        sel_idx = jnp.concatenate(idxs, axis=0)
        ex = jnp.exp(sel_vals - sel_vals[0:1])
        w = ex / jnp.sum(ex, axis=0, keepdims=True)

        sel_ref[0, :, pl.ds(h * hw, hw)] = sel_idx
        w_ref[0, :, pl.ds(h * hw, hw)] = w

        sum_p = jnp.sum(raw, axis=0)
        sumsq = jnp.sum(raw * raw, axis=0)
        m = sum_p * (1.0 / P)
        var = (sumsq - P * m * m) * (1.0 / (P - 1))
        term2 = (1.0 - jnp.sqrt(var)) ** 2
        part = part + ((OFF_BIAS * jnp.sum(m * m) + OFF_VAR * jnp.sum(term2))
                       * (1.0 / n_tok)).reshape(1, 1)

    @pl.when((b == 0) & (t == 0))
    def _():
        loss_ref[...] = jnp.zeros_like(loss_ref)

    loss_ref[...] += part


def kernel(tensor, keys, biases, partitions, connectome_biases):
    del partitions, connectome_biases
    B, T, _ = tensor.shape
    tb = 2048
    nb = T // tb
    n_tok = B * T

    sel, w, loss = pl.pallas_call(
        functools.partial(_fused_kernel, n_tok=n_tok, n_half=2),
        grid=(B, nb),
        in_specs=[
            pl.BlockSpec((1, tb, D), lambda b, t: (b, t, 0)),
            pl.BlockSpec((P, D), lambda b, t: (0, 0)),
            pl.BlockSpec((P,), lambda b, t: (0,)),
        ],
        out_specs=[
            pl.BlockSpec((1, KSEL, tb), lambda b, t: (b, 0, t)),
            pl.BlockSpec((1, KSEL, tb), lambda b, t: (b, 0, t)),
            pl.BlockSpec((1, 1), lambda b, t: (0, 0)),
        ],
        out_shape=[
            jax.ShapeDtypeStruct((B, KSEL, T), jnp.int32),
            jax.ShapeDtypeStruct((B, KSEL, T), jnp.float32),
            jax.ShapeDtypeStruct((1, 1), jnp.float32),
        ],
        compiler_params=pltpu.CompilerParams(
            dimension_semantics=("arbitrary", "arbitrary"),
        ),
    )(tensor, keys, biases)

    return sel, w, loss[0, 0]


# speedup vs baseline: 1.2706x; 1.0029x over previous
import functools

import jax
import jax.numpy as jnp
from jax.experimental import pallas as pl
from jax.experimental.pallas import tpu as pltpu

P = 32
KSEL = 8
D = 2048
OFF_BIAS = 0.01
OFF_VAR = 0.01


def _fused_kernel(x_ref, keys_ref, biases_ref, sel_ref, w_ref, loss_ref, *,
                  n_tok, n_half):
    b = pl.program_id(0)
    t = pl.program_id(1)

    keys = keys_ref[...]
    biases = biases_ref[...].reshape(P, 1)
    tb = x_ref.shape[1]
    hw = tb // n_half

    part = jnp.zeros((1, 1), jnp.float32)
    for h in range(n_half):
        x = x_ref[0, pl.ds(h * hw, hw), :]
        raw = jax.lax.dot_general(
            keys, x, (((1,), (1,)), ((), ())),
            preferred_element_type=jnp.float32,
        )
        scores = raw + biases

        row_ids = jax.lax.broadcasted_iota(jnp.int32, (P, hw), 0)
        work = scores
        vals = []
        idxs = []
        for _ in range(KSEL):
            mx = jnp.max(work, axis=0, keepdims=True)
            hit = work == mx
            idx = jnp.min(jnp.where(hit, row_ids, P), axis=0, keepdims=True)
            vals.append(mx)
            idxs.append(idx)
            work = jnp.where(row_ids == idx, -jnp.inf, work)

        sel_vals = jnp.concatenate(vals, axis=0)
        sel_idx = jnp.concatenate(idxs, axis=0)
        ex = jnp.exp(sel_vals - sel_vals[0:1])
        w = ex / jnp.sum(ex, axis=0, keepdims=True)

        sel_ref[0, :, pl.ds(h * hw, hw)] = sel_idx
        w_ref[0, :, pl.ds(h * hw, hw)] = w

        sum_p = jnp.sum(raw, axis=0)
        sumsq = jnp.sum(raw * raw, axis=0)
        m = sum_p * (1.0 / P)
        var = (sumsq - P * m * m) * (1.0 / (P - 1))
        term2 = (1.0 - jnp.sqrt(var)) ** 2
        part = part + ((OFF_BIAS * jnp.sum(m * m) + OFF_VAR * jnp.sum(term2))
                       * (1.0 / n_tok)).reshape(1, 1)

    @pl.when((b == 0) & (t == 0))
    def _():
        loss_ref[...] = jnp.zeros_like(loss_ref)

    loss_ref[...] += part


def kernel(tensor, keys, biases, partitions, connectome_biases):
    del partitions, connectome_biases
    B, T, _ = tensor.shape
    tb = 2048
    nb = T // tb
    n_tok = B * T

    sel, w, loss = pl.pallas_call(
        functools.partial(_fused_kernel, n_tok=n_tok, n_half=4),
        grid=(B, nb),
        in_specs=[
            pl.BlockSpec((1, tb, D), lambda b, t: (b, t, 0)),
            pl.BlockSpec((P, D), lambda b, t: (0, 0)),
            pl.BlockSpec((P,), lambda b, t: (0,)),
        ],
        out_specs=[
            pl.BlockSpec((1, KSEL, tb), lambda b, t: (b, 0, t)),
            pl.BlockSpec((1, KSEL, tb), lambda b, t: (b, 0, t)),
            pl.BlockSpec((1, 1), lambda b, t: (0, 0)),
        ],
        out_shape=[
            jax.ShapeDtypeStruct((B, KSEL, T), jnp.int32),
            jax.ShapeDtypeStruct((B, KSEL, T), jnp.float32),
            jax.ShapeDtypeStruct((1, 1), jnp.float32),
        ],
        compiler_params=pltpu.CompilerParams(
            dimension_semantics=("arbitrary", "arbitrary"),
        ),
    )(tensor, keys, biases)

    return sel, w, loss[0, 0]


# speedup vs baseline: 1.2725x; 1.0015x over previous
import functools

import jax
import jax.numpy as jnp
from jax.experimental import pallas as pl
from jax.experimental.pallas import tpu as pltpu

P = 32
KSEL = 8
D = 2048
OFF_BIAS = 0.01
OFF_VAR = 0.01


def _fused_kernel(x_ref, keys_ref, biases_ref, sel_ref, w_ref, loss_ref, *,
                  n_tok, n_half):
    b = pl.program_id(0)
    t = pl.program_id(1)

    keys = keys_ref[...]
    biases = biases_ref[...].reshape(P, 1)
    tb = x_ref.shape[1]
    hw = tb // n_half

    part = jnp.zeros((1, 1), jnp.float32)
    for h in range(n_half):
        x = x_ref[0, pl.ds(h * hw, hw), :]
        raw = jax.lax.dot_general(
            keys, x, (((1,), (1,)), ((), ())),
            preferred_element_type=jnp.float32,
        )
        scores = raw + biases

        row_ids = jax.lax.broadcasted_iota(jnp.int32, (P, hw), 0)
        work = scores
        vals = []
        idxs = []
        for _ in range(KSEL):
            mx = jnp.max(work, axis=0, keepdims=True)
            hit = work == mx
            idx = jnp.min(jnp.where(hit, row_ids, P), axis=0, keepdims=True)
            vals.append(mx)
            idxs.append(idx)
            work = jnp.where(row_ids == idx, -jnp.inf, work)

        sel_vals = jnp.concatenate(vals, axis=0)
        sel_idx = jnp.concatenate(idxs, axis=0)
        ex = jnp.exp(sel_vals - sel_vals[0:1])
        w = ex / jnp.sum(ex, axis=0, keepdims=True)

        sel_ref[0, :, pl.ds(h * hw, hw)] = sel_idx
        w_ref[0, :, pl.ds(h * hw, hw)] = w

        sum_p = jnp.sum(raw, axis=0)
        sumsq = jnp.sum(raw * raw, axis=0)
        m = sum_p * (1.0 / P)
        var = (sumsq - P * m * m) * (1.0 / (P - 1))
        term2 = (1.0 - jnp.sqrt(var)) ** 2
        part = part + ((OFF_BIAS * jnp.sum(m * m) + OFF_VAR * jnp.sum(term2))
                       * (1.0 / n_tok)).reshape(1, 1)

    @pl.when((b == 0) & (t == 0))
    def _():
        loss_ref[...] = jnp.zeros_like(loss_ref)

    loss_ref[...] += part


def kernel(tensor, keys, biases, partitions, connectome_biases):
    del partitions, connectome_biases
    B, T, _ = tensor.shape
    tb = 2048
    nb = T // tb
    n_tok = B * T

    sel, w, loss = pl.pallas_call(
        functools.partial(_fused_kernel, n_tok=n_tok, n_half=8),
        grid=(B, nb),
        in_specs=[
            pl.BlockSpec((1, tb, D), lambda b, t: (b, t, 0)),
            pl.BlockSpec((P, D), lambda b, t: (0, 0)),
            pl.BlockSpec((P,), lambda b, t: (0,)),
        ],
        out_specs=[
            pl.BlockSpec((1, KSEL, tb), lambda b, t: (b, 0, t)),
            pl.BlockSpec((1, KSEL, tb), lambda b, t: (b, 0, t)),
            pl.BlockSpec((1, 1), lambda b, t: (0, 0)),
        ],
        out_shape=[
            jax.ShapeDtypeStruct((B, KSEL, T), jnp.int32),
            jax.ShapeDtypeStruct((B, KSEL, T), jnp.float32),
            jax.ShapeDtypeStruct((1, 1), jnp.float32),
        ],
        compiler_params=pltpu.CompilerParams(
            dimension_semantics=("arbitrary", "arbitrary"),
        ),
    )(tensor, keys, biases)

    return sel, w, loss[0, 0]
